# baseline (speedup 1.0000x reference)
import functools

import jax
import jax.numpy as jnp
from jax import lax
from jax.experimental import pallas as pl
from jax.experimental.pallas import tpu as pltpu
from jax.experimental.pallas import tpu_sc as plsc

_R = 1024
_C = 100000
_K = 200
_NW = 32
_EPW = _R * _K // _NW
_CH = 128
_NCH = _EPW // _CH
_L = 16
_RPW = _R // _NW
_COLS = tuple(range(0, _K - _L, _L)) + (_K - _L,)


@functools.partial(
    pl.kernel,
    mesh=plsc.VectorSubcoreMesh(core_axis_name="c", subcore_axis_name="s"),
    out_type=jax.ShapeDtypeStruct((_NW, _NCH, _CH), jnp.float32),
    scratch_types=[
        pltpu.VMEM((_RPW, _K), jnp.int32),
        pltpu.VMEM((_EPW,), jnp.int32),
        pltpu.VMEM((_NCH, _CH), jnp.float32),
        pltpu.SemaphoreType.DMA,
    ],
)
def _sc_gather(tbl_hbm, idx_hbm, out_hbm, idx_v, pidx_v, out_v, sem):
    cid = lax.axis_index("c")
    sid = lax.axis_index("s")
    wid = sid * 2 + cid
    row0 = wid * _RPW

    pltpu.sync_copy(idx_hbm.at[pl.ds(row0, _RPW)], idx_v)

    def conv(r, carry):
        i = row0 + r
        sbase = ((i >> 7) << 10) + (i & 127)
        for c in _COLS:
            q = idx_v[r, pl.ds(c, _L)]
            p = ((q >> 3) << 13) + ((q & 7) << 7) + sbase
            pidx_v[pl.ds(r * _K + c, _L)] = p
        return carry

    lax.fori_loop(0, _RPW, conv, 0)

    def fire(g, carry):
        pltpu.async_copy(
            tbl_hbm.at[pidx_v.at[pl.ds(g * _CH, _CH)]], out_v.at[g], sem)
        return carry

    lax.fori_loop(0, _NCH, fire, 0)

    def drain(g, carry):
        pltpu.make_async_copy(
            tbl_hbm.at[pidx_v.at[pl.ds(g * _CH, _CH)]], out_v.at[g], sem).wait()
        return carry

    lax.fori_loop(0, _NCH, drain, 0)

    pltpu.sync_copy(out_v, out_hbm.at[wid])


def kernel(input_tensor, indices):
    tbl = (input_tensor.T.reshape(_C // 8, 8, _R // 128, 128)
           .transpose(0, 2, 1, 3).reshape(-1))
    out = _sc_gather(tbl, indices)
    return out.reshape(_R, _K)


# speedup vs baseline: 1.0571x; 1.0571x over previous
import functools

import jax
import jax.numpy as jnp
from jax import lax
from jax.experimental import pallas as pl
from jax.experimental.pallas import tpu as pltpu
from jax.experimental.pallas import tpu_sc as plsc

_R = 1024
_C = 100000
_K = 200
_NW = 32
_EPW = _R * _K // _NW
_CH = 128
_NCH = _EPW // _CH
_L = 16
_RPW = _R // _NW
_MAGIC = 335545
_SHIFT = 26


@functools.partial(
    pl.kernel,
    mesh=plsc.VectorSubcoreMesh(core_axis_name="c", subcore_axis_name="s"),
    out_type=jax.ShapeDtypeStruct((_NW, _NCH, _CH), jnp.float32),
    scratch_types=[
        pltpu.VMEM((_NCH, _CH), jnp.int32),
        pltpu.VMEM((_NCH, _CH), jnp.float32),
        pltpu.SemaphoreType.DMA,
    ],
)
def _sc_gather(tbl_hbm, idx_hbm, out_hbm, idx_v, out_v, sem):
    cid = lax.axis_index("c")
    sid = lax.axis_index("s")
    wid = sid * 2 + cid

    pltpu.sync_copy(idx_hbm.at[wid], idx_v)

    row0 = wid * _RPW

    def conv_fire(r, carry):
        for u in range(_CH // _L):
            sl = pl.ds(u * _L, _L)
            n = (r * _CH + u * _L) + lax.iota(jnp.int32, _L)
            i = row0 + ((n * _MAGIC) >> _SHIFT)
            q = idx_v[r, sl]
            idx_v[r, sl] = (((q >> 3) << 13) + ((i >> 7) << 10)
                            + ((q & 7) << 7) + (i & 127))
        pltpu.async_copy(tbl_hbm.at[idx_v.at[r]], out_v.at[r], sem)
        return carry

    lax.fori_loop(0, _NCH, conv_fire, 0)

    def drain(r, carry):
        pltpu.make_async_copy(tbl_hbm.at[idx_v.at[r]], out_v.at[r], sem).wait()
        return carry

    lax.fori_loop(0, _NCH, drain, 0)

    pltpu.sync_copy(out_v, out_hbm.at[wid])


def kernel(input_tensor, indices):
    tbl = (input_tensor.T.reshape(_C // 8, 8, _R // 128, 128)
           .transpose(0, 2, 1, 3).reshape(-1))
    idx3d = indices.reshape(_NW, _NCH, _CH)
    out = _sc_gather(tbl, idx3d)
    return out.reshape(_R, _K)


# speedup vs baseline: 1.1810x; 1.1172x over previous
import functools

import jax
import jax.numpy as jnp
from jax import lax
from jax.experimental import pallas as pl
from jax.experimental.pallas import tpu as pltpu
from jax.experimental.pallas import tpu_sc as plsc

_R = 1024
_C = 100000
_K = 200
_NW = 32
_L = 16
_NA = _K // 8
_NC = _R // 128
_AMAX = 7
_A0 = (0, 7, 13, 19)


@functools.partial(
    pl.kernel,
    mesh=plsc.VectorSubcoreMesh(core_axis_name="c", subcore_axis_name="s"),
    out_type=jax.ShapeDtypeStruct((_NA, _NC, 8, 128), jnp.float32),
    scratch_types=[
        pltpu.VMEM((_AMAX * 8 * 128,), jnp.int32),
        pltpu.VMEM((_AMAX * 8, 128), jnp.int32),
        pltpu.VMEM((_AMAX * 8, 128), jnp.float32),
        pltpu.SemaphoreType.DMA,
    ],
)
def _sc_gather(tbl_hbm, idx_hbm, out_hbm, idx_v, pidx_v, out_v, sem):
    cid = lax.axis_index("c")
    sid = lax.axis_index("s")
    wid = sid * 2 + cid
    c0 = wid & 7
    t = wid >> 3
    a0 = jnp.where(t == 0, 0, t * 6 + 1)
    na = jnp.where(t == 0, 7, 6)

    def stage(k, carry):
        pltpu.async_copy(
            idx_hbm.at[pl.ds(((a0 + k) * _NC + c0) * 1024, 1024)],
            idx_v.at[pl.ds(k * 1024, 1024)], sem)
        return carry

    lax.fori_loop(0, na, stage, 0)

    def stage_wait(k, carry):
        pltpu.make_async_copy(
            idx_hbm.at[pl.ds(((a0 + k) * _NC + c0) * 1024, 1024)],
            idx_v.at[pl.ds(k * 1024, 1024)], sem).wait()
        return carry

    lax.fori_loop(0, na, stage_wait, 0)

    def conv_fire(k, carry):
        for b in range(8):
            jl = k * 8 + b
            for v in range(128 // _L):
                sbase = (c0 << 10) + v * _L + lax.iota(jnp.int32, _L)
                q = idx_v[pl.ds(jl * 128 + v * _L, _L)]
                pidx_v[jl, pl.ds(v * _L, _L)] = (
                    ((q >> 3) << 13) + ((q & 7) << 7) + sbase)
            pltpu.async_copy(tbl_hbm.at[pidx_v.at[jl]], out_v.at[jl], sem)
        return carry

    lax.fori_loop(0, na, conv_fire, 0)

    def drain(jl, carry):
        pltpu.make_async_copy(
            tbl_hbm.at[pidx_v.at[jl]], out_v.at[jl], sem).wait()
        return carry

    lax.fori_loop(0, na * 8, drain, 0)

    def wb(k, carry):
        pltpu.sync_copy(out_v.at[pl.ds(k * 8, 8)], out_hbm.at[a0 + k, c0])
        return carry

    lax.fori_loop(0, na, wb, 0)


def kernel(input_tensor, indices):
    tbl = (input_tensor.T.reshape(_C // 8, 8, _R // 128, 128)
           .transpose(0, 2, 1, 3).reshape(-1))
    idx = (indices.T.reshape(_NA, 8, _NC, 128)
           .transpose(0, 2, 1, 3).reshape(-1))
    out = _sc_gather(tbl, idx)
    return out.transpose(0, 2, 1, 3).reshape(_K, _R).T
